# baseline (speedup 1.0000x reference)
import functools
import jax
import jax.numpy as jnp
from jax import lax
from jax.experimental import pallas as pl
from jax.experimental.pallas import tpu as pltpu
from jax.experimental.pallas import tpu_sc as plsc

_NC = 2
_NS = 16
_NW = _NC * _NS
_L = 16
_CH = 80
_BP = 64


def _sc_pool_body(m1_hbm, m2_hbm, seg_hbm, w1_hbm, w2_hbm,
                  out1, od1, out2, od2,
                  mv1, mv2, segv, w1v, w2v, n1, d1, n2, d2,
                  sem0, sem1, *, d, nchunks):
    wid = lax.axis_index("s") * _NC + lax.axis_index("c")
    kmax = (nchunks + _NW - 1) // _NW
    nj = d // _L

    pltpu.sync_copy(w1_hbm, w1v)
    pltpu.sync_copy(w2_hbm, w2v)

    zero16 = jnp.zeros((_L,), jnp.float32)

    @plsc.parallel_loop(0, (_BP * d) // _L, step=1, unroll=8)
    def _zacc(i):
        n1[pl.ds(i * _L, _L)] = zero16
        n2[pl.ds(i * _L, _L)] = zero16

    @plsc.parallel_loop(0, _BP, step=1, unroll=8)
    def _zden(i):
        d1[pl.ds(i * _L, _L)] = zero16
        d2[pl.ds(i * _L, _L)] = zero16

    w1r = [w1v[pl.ds(j * _L, _L)] for j in range(nj)]
    w2r = [w2v[pl.ds(j * _L, _L)] for j in range(nj)]

    lane = lax.broadcasted_iota(jnp.int32, (_L,), 0)
    bfly = [lane ^ sh for sh in (1, 2, 4, 8)]
    gdn = lax.GatherDimensionNumbers(
        offset_dims=(), collapsed_slice_dims=(0,), start_index_map=(0,))

    def lane_sum(v):
        for idx in bfly:
            v = v + lax.gather(
                v, idx[:, None], dimension_numbers=gdn, slice_sizes=(1,),
                mode=lax.GatherScatterMode.PROMISE_IN_BOUNDS)
        return v

    sems = (sem0, sem1)
    bufs1 = (mv1.at[0], mv1.at[1])
    bufs2 = (mv2.at[0], mv2.at[1])

    def start_chunk(c, bi):
        pltpu.async_copy(m1_hbm.at[pl.ds(c * _CH, _CH), :], bufs1[bi], sems[bi])
        pltpu.async_copy(m2_hbm.at[pl.ds(c * _CH, _CH), :], bufs2[bi], sems[bi])
        pltpu.async_copy(seg_hbm.at[pl.ds(c * _CH, _CH)],
                         segv.at[bi, pl.ds(0, _CH)], sems[bi])

    def wait_chunk(c, bi):
        pltpu.make_async_copy(
            m1_hbm.at[pl.ds(c * _CH, _CH), :], bufs1[bi], sems[bi]).wait()
        pltpu.make_async_copy(
            m2_hbm.at[pl.ds(c * _CH, _CH), :], bufs2[bi], sems[bi]).wait()
        pltpu.make_async_copy(
            seg_hbm.at[pl.ds(c * _CH, _CH)],
            segv.at[bi, pl.ds(0, _CH)], sems[bi]).wait()

    @pl.when(wid < nchunks)
    def _prime():
        start_chunk(wid, 0)

    def chunk_pair(k2, carry):
        for bi in (0, 1):
            c = wid + (k2 * 2 + bi) * _NW

            @pl.when(c + _NW < nchunks)
            def _prefetch():
                start_chunk(c + _NW, bi ^ 1)

            @pl.when(c < nchunks)
            def _process():
                wait_chunk(c, bi)

                @plsc.parallel_loop(0, _CH, step=1, unroll=4)
                def _rows(r):
                    s = segv[bi, pl.ds(r, _L)][0]
                    for mv, wr, nacc, dacc in ((bufs1[bi], w1r, n1, d1),
                                               (bufs2[bi], w2r, n2, d2)):
                        sl = [mv[r, pl.ds(j * _L, _L)] for j in range(nj)]
                        acc = sl[0] * wr[0]
                        for j in range(1, nj):
                            acc = acc + sl[j] * wr[j]
                        ev = jnp.exp(lane_sum(acc))
                        plsc.addupdate(dacc.at[pl.ds(s * _L, _L)], ev)
                        for j in range(nj):
                            plsc.addupdate(
                                nacc.at[pl.ds(s * d + j * _L, _L)],
                                sl[j] * ev)
        return carry
    lax.fori_loop(0, (kmax + 1) // 2, chunk_pair, 0)

    acc_sz = _BP * d
    den_sz = _BP * _L
    pltpu.sync_copy(n1, out1.at[pl.ds(wid * acc_sz, acc_sz)])
    pltpu.sync_copy(d1, od1.at[pl.ds(wid * den_sz, den_sz)])
    pltpu.sync_copy(n2, out2.at[pl.ds(wid * acc_sz, acc_sz)])
    pltpu.sync_copy(d2, od2.at[pl.ds(wid * den_sz, den_sz)])


def _att_body(q_ref, p1_ref, pd1_ref, p2_ref, pd2_ref, att_ref):
    q = q_ref[...]
    num1 = jnp.sum(p1_ref[...], axis=0)
    num2 = jnp.sum(p2_ref[...], axis=0)
    den1 = jnp.sum(pd1_ref[...], axis=0)[:, 0:1]
    den2 = jnp.sum(pd2_ref[...], axis=0)[:, 0:1]
    k1 = num1 / jnp.where(den1 > 0, den1, 1.0)
    k2 = num2 / jnp.where(den2 > 0, den2, 1.0)
    s1 = jnp.sum(q * k1, axis=1, keepdims=True)
    s2 = jnp.sum(q * k2, axis=1, keepdims=True)
    m = jnp.maximum(s1, s2)
    e1 = jnp.exp(s1 - m)
    e2 = jnp.exp(s2 - m)
    tot = e1 + e2
    att_ref[...] = jnp.concatenate([e1 / tot, e2 / tot], axis=1)


def _mix_body(seg_ref, m1_ref, m2_ref, att_ref, out_ref, *, bp):
    seg = seg_ref[0, 0, :]
    bn = seg.shape[0]
    oh = (seg[:, None] ==
          jax.lax.broadcasted_iota(jnp.int32, (bn, bp), 1)).astype(jnp.float32)
    attn = jnp.dot(oh, att_ref[...], preferred_element_type=jnp.float32)
    out_ref[...] = (m1_ref[...] * attn[:, 0:1] + m2_ref[...] * attn[:, 1:2])


def kernel(Q, M1, M2, segment_ids, W1, b1, W2, b2):
    del b1, b2
    n, d = M1.shape
    b = Q.shape[0]
    bp = _BP
    assert n % _CH == 0 and d % _L == 0 and b <= bp
    nchunks = n // _CH
    f32 = jnp.float32

    mesh = plsc.VectorSubcoreMesh(core_axis_name="c", subcore_axis_name="s",
                                  num_cores=_NC, num_subcores=_NS)
    sc_pool = pl.kernel(
        functools.partial(_sc_pool_body, d=d, nchunks=nchunks),
        out_type=[
            jax.ShapeDtypeStruct((_NW * bp * d,), f32),
            jax.ShapeDtypeStruct((_NW * bp * _L,), f32),
            jax.ShapeDtypeStruct((_NW * bp * d,), f32),
            jax.ShapeDtypeStruct((_NW * bp * _L,), f32),
        ],
        mesh=mesh,
        scratch_types=[
            pltpu.VMEM((2, _CH, d), f32),
            pltpu.VMEM((2, _CH, d), f32),
            pltpu.VMEM((2, _CH + _L), jnp.int32),
            pltpu.VMEM((d,), f32),
            pltpu.VMEM((d,), f32),
            pltpu.VMEM((bp * d,), f32),
            pltpu.VMEM((bp * _L,), f32),
            pltpu.VMEM((bp * d,), f32),
            pltpu.VMEM((bp * _L,), f32),
            pltpu.SemaphoreType.DMA,
            pltpu.SemaphoreType.DMA,
        ],
        compiler_params=pltpu.CompilerParams(use_tc_tiling_on_sc=True),
    )
    out1, od1, out2, od2 = sc_pool(
        M1, M2, segment_ids, W1.reshape(-1), W2.reshape(-1))
    p1 = out1.reshape(_NW, bp, d)
    pd1 = od1.reshape(_NW, bp, _L)
    p2 = out2.reshape(_NW, bp, d)
    pd2 = od2.reshape(_NW, bp, _L)

    qp = jnp.zeros((bp, d), f32).at[:b].set(Q)
    att = pl.pallas_call(
        _att_body,
        out_shape=jax.ShapeDtypeStruct((bp, 2), f32),
    )(qp, p1, pd1, p2, pd2)

    bm = 5000
    nm = n // bm
    assert nm * bm == n
    seg3m = segment_ids.reshape(nm, 1, bm)

    out = pl.pallas_call(
        functools.partial(_mix_body, bp=bp),
        grid=(nm,),
        in_specs=[
            pl.BlockSpec((1, 1, bm), lambda i: (i, 0, 0)),
            pl.BlockSpec((bm, d), lambda i: (i, 0)),
            pl.BlockSpec((bm, d), lambda i: (i, 0)),
            pl.BlockSpec((bp, 2), lambda i: (0, 0)),
        ],
        out_specs=pl.BlockSpec((bm, d), lambda i: (i, 0)),
        out_shape=jax.ShapeDtypeStruct((n, d), f32),
    )(seg3m, M1, M2, att)
    return out


# speedup vs baseline: 1.2379x; 1.2379x over previous
import functools
import jax
import jax.numpy as jnp
from jax import lax
from jax.experimental import pallas as pl
from jax.experimental.pallas import tpu as pltpu
from jax.experimental.pallas import tpu_sc as plsc

_NC = 2
_NS = 16
_NW = _NC * _NS
_L = 16
_CH = 80
_BP = 64


def _sc_pool_body(m1_hbm, m2_hbm, seg_hbm, w1_hbm, w2_hbm,
                  out1, od1, out2, od2,
                  mv1, mv2, segv, w1v, w2v, n1, d1, n2, d2,
                  sem0, sem1, *, d, nchunks):
    wid = lax.axis_index("s") * _NC + lax.axis_index("c")
    kmax = (nchunks + _NW - 1) // _NW
    nj = d // _L

    pltpu.sync_copy(w1_hbm, w1v)
    pltpu.sync_copy(w2_hbm, w2v)

    zero16 = jnp.zeros((_L,), jnp.float32)

    @plsc.parallel_loop(0, (_BP * d) // _L, step=1, unroll=8)
    def _zacc(i):
        n1[pl.ds(i * _L, _L)] = zero16
        n2[pl.ds(i * _L, _L)] = zero16

    @plsc.parallel_loop(0, _BP, step=1, unroll=8)
    def _zden(i):
        d1[pl.ds(i * _L, _L)] = zero16
        d2[pl.ds(i * _L, _L)] = zero16

    w1r = [w1v[pl.ds(j * _L, _L)] for j in range(nj)]
    w2r = [w2v[pl.ds(j * _L, _L)] for j in range(nj)]

    lane = lax.broadcasted_iota(jnp.int32, (_L,), 0)
    bfly = [lane ^ sh for sh in (1, 2, 4, 8)]
    gdn = lax.GatherDimensionNumbers(
        offset_dims=(), collapsed_slice_dims=(0,), start_index_map=(0,))

    def lane_sum(v):
        for idx in bfly:
            v = v + lax.gather(
                v, idx[:, None], dimension_numbers=gdn, slice_sizes=(1,),
                mode=lax.GatherScatterMode.PROMISE_IN_BOUNDS)
        return v

    sems = (sem0, sem1)
    bufs1 = (mv1.at[0], mv1.at[1])
    bufs2 = (mv2.at[0], mv2.at[1])

    def start_chunk(c, bi):
        pltpu.async_copy(m1_hbm.at[pl.ds(c * _CH, _CH), :], bufs1[bi], sems[bi])
        pltpu.async_copy(m2_hbm.at[pl.ds(c * _CH, _CH), :], bufs2[bi], sems[bi])
        pltpu.async_copy(seg_hbm.at[pl.ds(c * _CH, _CH)],
                         segv.at[bi, pl.ds(0, _CH)], sems[bi])

    def wait_chunk(c, bi):
        pltpu.make_async_copy(
            m1_hbm.at[pl.ds(c * _CH, _CH), :], bufs1[bi], sems[bi]).wait()
        pltpu.make_async_copy(
            m2_hbm.at[pl.ds(c * _CH, _CH), :], bufs2[bi], sems[bi]).wait()
        pltpu.make_async_copy(
            seg_hbm.at[pl.ds(c * _CH, _CH)],
            segv.at[bi, pl.ds(0, _CH)], sems[bi]).wait()

    @pl.when(wid < nchunks)
    def _prime():
        start_chunk(wid, 0)

    def chunk_pair(k2, carry):
        for bi in (0, 1):
            c = wid + (k2 * 2 + bi) * _NW

            @pl.when(c + _NW < nchunks)
            def _prefetch():
                start_chunk(c + _NW, bi ^ 1)

            @pl.when(c < nchunks)
            def _process():
                wait_chunk(c, bi)

                @plsc.parallel_loop(0, _CH, step=1, unroll=2)
                def _rows(r):
                    s = segv[bi, pl.ds(r, _L)][0]
                    for mv, wr, nacc, dacc in ((bufs1[bi], w1r, n1, d1),
                                               (bufs2[bi], w2r, n2, d2)):
                        sl = [mv[r, pl.ds(j * _L, _L)] for j in range(nj)]
                        acc = sl[0] * wr[0]
                        for j in range(1, nj):
                            acc = acc + sl[j] * wr[j]
                        ev = jnp.exp(lane_sum(acc))
                        plsc.addupdate(dacc.at[pl.ds(s * _L, _L)], ev)
                        for j in range(nj):
                            plsc.addupdate(
                                nacc.at[pl.ds(s * d + j * _L, _L)],
                                sl[j] * ev)
        return carry
    lax.fori_loop(0, (kmax + 1) // 2, chunk_pair, 0)

    acc_sz = _BP * d
    den_sz = _BP * _L
    pltpu.sync_copy(n1, out1.at[pl.ds(wid * acc_sz, acc_sz)])
    pltpu.sync_copy(d1, od1.at[pl.ds(wid * den_sz, den_sz)])
    pltpu.sync_copy(n2, out2.at[pl.ds(wid * acc_sz, acc_sz)])
    pltpu.sync_copy(d2, od2.at[pl.ds(wid * den_sz, den_sz)])


def _att_body(q_ref, p1_ref, pd1_ref, p2_ref, pd2_ref, att_ref):
    q = q_ref[...]
    num1 = jnp.sum(p1_ref[...], axis=0)
    num2 = jnp.sum(p2_ref[...], axis=0)
    den1 = jnp.sum(pd1_ref[...], axis=0)[:, 0:1]
    den2 = jnp.sum(pd2_ref[...], axis=0)[:, 0:1]
    k1 = num1 / jnp.where(den1 > 0, den1, 1.0)
    k2 = num2 / jnp.where(den2 > 0, den2, 1.0)
    s1 = jnp.sum(q * k1, axis=1, keepdims=True)
    s2 = jnp.sum(q * k2, axis=1, keepdims=True)
    m = jnp.maximum(s1, s2)
    e1 = jnp.exp(s1 - m)
    e2 = jnp.exp(s2 - m)
    tot = e1 + e2
    att_ref[...] = jnp.concatenate([e1 / tot, e2 / tot], axis=1)


def _mix_body(seg_ref, m1_ref, m2_ref, att_ref, out_ref, *, bp):
    seg = seg_ref[0, 0, :]
    bn = seg.shape[0]
    oh = (seg[:, None] ==
          jax.lax.broadcasted_iota(jnp.int32, (bn, bp), 1)).astype(jnp.float32)
    attn = jnp.dot(oh, att_ref[...], preferred_element_type=jnp.float32)
    out_ref[...] = (m1_ref[...] * attn[:, 0:1] + m2_ref[...] * attn[:, 1:2])


def kernel(Q, M1, M2, segment_ids, W1, b1, W2, b2):
    del b1, b2
    n, d = M1.shape
    b = Q.shape[0]
    bp = _BP
    assert n % _CH == 0 and d % _L == 0 and b <= bp
    nchunks = n // _CH
    f32 = jnp.float32

    mesh = plsc.VectorSubcoreMesh(core_axis_name="c", subcore_axis_name="s",
                                  num_cores=_NC, num_subcores=_NS)
    sc_pool = pl.kernel(
        functools.partial(_sc_pool_body, d=d, nchunks=nchunks),
        out_type=[
            jax.ShapeDtypeStruct((_NW * bp * d,), f32),
            jax.ShapeDtypeStruct((_NW * bp * _L,), f32),
            jax.ShapeDtypeStruct((_NW * bp * d,), f32),
            jax.ShapeDtypeStruct((_NW * bp * _L,), f32),
        ],
        mesh=mesh,
        scratch_types=[
            pltpu.VMEM((2, _CH, d), f32),
            pltpu.VMEM((2, _CH, d), f32),
            pltpu.VMEM((2, _CH + _L), jnp.int32),
            pltpu.VMEM((d,), f32),
            pltpu.VMEM((d,), f32),
            pltpu.VMEM((bp * d,), f32),
            pltpu.VMEM((bp * _L,), f32),
            pltpu.VMEM((bp * d,), f32),
            pltpu.VMEM((bp * _L,), f32),
            pltpu.SemaphoreType.DMA,
            pltpu.SemaphoreType.DMA,
        ],
        compiler_params=pltpu.CompilerParams(use_tc_tiling_on_sc=True),
    )
    out1, od1, out2, od2 = sc_pool(
        M1, M2, segment_ids, W1.reshape(-1), W2.reshape(-1))
    p1 = out1.reshape(_NW, bp, d)
    pd1 = od1.reshape(_NW, bp, _L)
    p2 = out2.reshape(_NW, bp, d)
    pd2 = od2.reshape(_NW, bp, _L)

    qp = jnp.zeros((bp, d), f32).at[:b].set(Q)
    att = pl.pallas_call(
        _att_body,
        out_shape=jax.ShapeDtypeStruct((bp, 2), f32),
    )(qp, p1, pd1, p2, pd2)

    bm = 5000
    nm = n // bm
    assert nm * bm == n
    seg3m = segment_ids.reshape(nm, 1, bm)

    out = pl.pallas_call(
        functools.partial(_mix_body, bp=bp),
        grid=(nm,),
        in_specs=[
            pl.BlockSpec((1, 1, bm), lambda i: (i, 0, 0)),
            pl.BlockSpec((bm, d), lambda i: (i, 0)),
            pl.BlockSpec((bm, d), lambda i: (i, 0)),
            pl.BlockSpec((bp, 2), lambda i: (0, 0)),
        ],
        out_specs=pl.BlockSpec((bm, d), lambda i: (i, 0)),
        out_shape=jax.ShapeDtypeStruct((n, d), f32),
    )(seg3m, M1, M2, att)
    return out
